# Initial kernel scaffold; baseline (speedup 1.0000x reference)
import jax
import jax.numpy as jnp
from jax.experimental import pallas as pl


def kernel(tok_seq, tok_type_ids, W_word, W_type, W_rel):
    raise NotImplementedError("write your pallas kernel here")


# speedup vs baseline: 3.3294x; 3.3294x over previous
import functools

import jax
import jax.numpy as jnp
from jax import lax
from jax.experimental import pallas as pl
from jax.experimental.pallas import tpu as pltpu
from jax.experimental.pallas import tpu_sc as plsc

_VOCAB = 100000
_D = 64
_MAX_OFF = 128
_EPS = 1e-12
_B, _S = 4, 512

_TAB_ROWS = 1040
_KPAD = 264


def _layernorm_rows(x):
    u = jnp.mean(x, axis=-1, keepdims=True)
    s = jnp.mean((x - u) ** 2, axis=-1, keepdims=True)
    return (x - u) / jnp.sqrt(s + _EPS)


def _embd2_body(wrel_ref, types_smem, tcol_ref, out_ref, tab_s):
    b = pl.program_id(0)
    i = pl.program_id(1)

    @pl.when((b == 0) & (i == 0))
    def _build_table():
        kk = lax.broadcasted_iota(jnp.int32, (_TAB_ROWS, _KPAD), 0)
        vv = lax.broadcasted_iota(jnp.int32, (_TAB_ROWS, _KPAD), 1)
        d = kk - (_S - 1)
        toepval = jnp.where(d >= 0, jnp.minimum(d, _MAX_OFF - 1),
                            jnp.maximum(d, -_MAX_OFF + 1) + 2 * _MAX_OFF)
        g = jnp.where(kk < 2 * _S - 1, toepval, 0)
        g = jnp.where(kk == 1024, _MAX_OFF, g)
        g = jnp.where(kk == 1025, 2 * _MAX_OFF, g)
        g = jnp.where(kk == 1026, 2 * _MAX_OFF + 1, g)
        onehot = (g == vv).astype(jnp.float32)
        ln_rel = _layernorm_rows(wrel_ref[...])
        tab_s[...] = jnp.dot(onehot, ln_rel,
                             precision=lax.Precision.HIGHEST,
                             preferred_element_type=jnp.float32)

    ti = types_smem[b, i]
    mask = tcol_ref[0] == ti
    win = tab_s[pl.ds(_S - 1 - i, _S), :]
    r128 = tab_s[1024:1025, :]
    base = jnp.where(mask, win, r128)

    @pl.when(i == 0)
    def _first_row():
        jcol = lax.broadcasted_iota(jnp.int32, (_S, 1), 0)
        out_ref[0, 0] = jnp.where(jcol >= 1, tab_s[1025:1026, :], base)

    @pl.when(i > 0)
    def _other_rows():
        out_ref[0, 0] = base
        out_ref[0, 0, 0:1, :] = tab_s[1026:1027, :]


def _embd1_body(rows_ref, tcol_ref, wtype_ref, out_ref):
    mask = tcol_ref[0] == 0
    tw = jnp.where(mask, wtype_ref[0:1, :], wtype_ref[1:2, :])
    out_ref[0] = _layernorm_rows(rows_ref[0] + tw)


_SC_CORES = 2
_SC_SUBCORES = 16
_NW = _SC_CORES * _SC_SUBCORES
_ROWS_PER_W = (_B * _S) // _NW


def _sc_word_gather_body(table_hbm, idx_hbm, out_hbm, idx_v, rows_v, sem):
    wid = lax.axis_index("s") * _SC_CORES + lax.axis_index("c")
    base = wid * _ROWS_PER_W
    pltpu.sync_copy(idx_hbm.at[pl.ds(base, _ROWS_PER_W)], idx_v)
    pltpu.async_copy(table_hbm.at[idx_v], rows_v, sem).wait()
    pltpu.sync_copy(rows_v, out_hbm.at[pl.ds(base, _ROWS_PER_W)])


def _make_sc_word_gather():
    return pl.kernel(
        _sc_word_gather_body,
        mesh=plsc.VectorSubcoreMesh(core_axis_name="c", subcore_axis_name="s"),
        out_type=jax.ShapeDtypeStruct((_B * _S, _D), jnp.float32),
        scratch_types=[
            pltpu.VMEM((_ROWS_PER_W,), jnp.int32),
            pltpu.VMEM((_ROWS_PER_W, _D), jnp.float32),
            pltpu.SemaphoreType.DMA,
        ],
        compiler_params=pltpu.CompilerParams(use_tc_tiling_on_sc=False),
    )


def kernel(tok_seq, tok_type_ids, W_word, W_type, W_rel):
    types_col = tok_type_ids[:, :, None]
    wrel_pad = jnp.zeros((_KPAD, _D), jnp.float32).at[: 2 * _MAX_OFF + 2].set(W_rel)

    word_rows = _make_sc_word_gather()(W_word, tok_seq.reshape(-1))

    embd2 = pl.pallas_call(
        _embd2_body,
        grid=(_B, _S),
        in_specs=[
            pl.BlockSpec((_KPAD, _D), lambda b, i: (0, 0)),
            pl.BlockSpec(memory_space=pltpu.SMEM),
            pl.BlockSpec((1, _S, 1), lambda b, i: (b, 0, 0)),
        ],
        out_specs=pl.BlockSpec((1, 1, _S, _D), lambda b, i: (b, i, 0, 0)),
        out_shape=jax.ShapeDtypeStruct((_B, _S, _S, _D), jnp.float32),
        scratch_shapes=[pltpu.VMEM((_TAB_ROWS, _D), jnp.float32)],
        compiler_params=pltpu.CompilerParams(
            dimension_semantics=("arbitrary", "arbitrary")),
    )(wrel_pad, tok_type_ids, types_col)

    embd1 = pl.pallas_call(
        _embd1_body,
        grid=(_B,),
        in_specs=[
            pl.BlockSpec((1, _S, _D), lambda b: (b, 0, 0)),
            pl.BlockSpec((1, _S, 1), lambda b: (b, 0, 0)),
            pl.BlockSpec((2, _D), lambda b: (0, 0)),
        ],
        out_specs=pl.BlockSpec((1, _S, _D), lambda b: (b, 0, 0)),
        out_shape=jax.ShapeDtypeStruct((_B, _S, _D), jnp.float32),
    )(word_rows.reshape(_B, _S, _D), types_col, W_type)

    return (embd1, embd2)


# speedup vs baseline: 5.7829x; 1.7369x over previous
import functools

import jax
import jax.numpy as jnp
from jax import lax
from jax.experimental import pallas as pl
from jax.experimental.pallas import tpu as pltpu
from jax.experimental.pallas import tpu_sc as plsc

_VOCAB = 100000
_D = 64
_MAX_OFF = 128
_EPS = 1e-12
_B, _S = 4, 512

_TAB_ROWS = 1040
_KPAD = 264
_BANKS = 8
_BI = 16


def _layernorm_rows(x):
    u = jnp.mean(x, axis=-1, keepdims=True)
    s = jnp.mean((x - u) ** 2, axis=-1, keepdims=True)
    return (x - u) / jnp.sqrt(s + _EPS)


def _embd2_body(wrel_ref, types_smem, tcol_ref, out_ref, tab_s):
    b = pl.program_id(0)
    ib = pl.program_id(1)

    @pl.when((b == 0) & (ib == 0))
    def _build_table():
        rr = lax.broadcasted_iota(jnp.int32, (_BANKS, _TAB_ROWS, _KPAD), 0)
        kk = lax.broadcasted_iota(jnp.int32, (_BANKS, _TAB_ROWS, _KPAD), 1)
        vv = lax.broadcasted_iota(jnp.int32, (_BANKS, _TAB_ROWS, _KPAD), 2)
        karg = kk + rr
        d = karg - (_S - 1)
        toepval = jnp.where(d >= 0, jnp.minimum(d, _MAX_OFF - 1),
                            jnp.maximum(d, -_MAX_OFF + 1) + 2 * _MAX_OFF)
        g = jnp.where(karg < 2 * _S - 1, toepval, 0)
        g = jnp.where(karg == 1024, _MAX_OFF, g)
        g = jnp.where(karg == 1025, 2 * _MAX_OFF, g)
        g = jnp.where(karg == 1026, 2 * _MAX_OFF + 1, g)
        onehot = (g == vv).astype(jnp.float32)
        onehot = onehot.reshape(_BANKS * _TAB_ROWS, _KPAD)
        ln_rel = _layernorm_rows(wrel_ref[...])
        tab_s[...] = jnp.dot(onehot, ln_rel,
                             precision=lax.Precision.HIGHEST,
                             preferred_element_type=jnp.float32)

    r128 = tab_s[1024:1025, :]
    m0 = tcol_ref[0] == 0

    def _store_base(c, base):
        if c == 0:
            @pl.when(ib == 0)
            def _first_row():
                jcol = lax.broadcasted_iota(jnp.int32, (_S, 1), 0)
                out_ref[0, 0] = jnp.where(jcol >= 1, tab_s[1025:1026, :], base)

            @pl.when(ib > 0)
            def _other_first():
                out_ref[0, 0] = base
                out_ref[0, 0, 0:1, :] = tab_s[1026:1027, :]
        else:
            out_ref[0, c] = base
            out_ref[0, c, 0:1, :] = tab_s[1026:1027, :]

    for c in range(_BI):
        i = ib * _BI + c
        ti = types_smem[b, i]
        s = _S - 1 - i
        r = lax.rem(s, 8)
        start = pl.multiple_of(r * _TAB_ROWS + (s - r), 8)
        win = tab_s[pl.ds(start, _S), :]

        @pl.when(ti == 0)
        def _type0(c=c, win=win):
            _store_base(c, jnp.where(m0, win, r128))

        @pl.when(ti != 0)
        def _type1(c=c, win=win):
            _store_base(c, jnp.where(m0, r128, win))


def _embd1_body(rows_ref, tcol_ref, wtype_ref, out_ref):
    mask = tcol_ref[0] == 0
    tw = jnp.where(mask, wtype_ref[0:1, :], wtype_ref[1:2, :])
    out_ref[0] = _layernorm_rows(rows_ref[0] + tw)


_SC_CORES = 2
_SC_SUBCORES = 16
_NW = _SC_CORES * _SC_SUBCORES
_ROWS_PER_W = (_B * _S) // _NW


def _sc_word_gather_body(table_hbm, idx_hbm, out_hbm, idx_v, rows_v, sem):
    wid = lax.axis_index("s") * _SC_CORES + lax.axis_index("c")
    base = wid * _ROWS_PER_W
    pltpu.sync_copy(idx_hbm.at[pl.ds(base, _ROWS_PER_W)], idx_v)
    pltpu.async_copy(table_hbm.at[idx_v], rows_v, sem).wait()
    pltpu.sync_copy(rows_v, out_hbm.at[pl.ds(base, _ROWS_PER_W)])


def _make_sc_word_gather():
    return pl.kernel(
        _sc_word_gather_body,
        mesh=plsc.VectorSubcoreMesh(core_axis_name="c", subcore_axis_name="s"),
        out_type=jax.ShapeDtypeStruct((_B * _S, _D), jnp.float32),
        scratch_types=[
            pltpu.VMEM((_ROWS_PER_W,), jnp.int32),
            pltpu.VMEM((_ROWS_PER_W, _D), jnp.float32),
            pltpu.SemaphoreType.DMA,
        ],
        compiler_params=pltpu.CompilerParams(use_tc_tiling_on_sc=False),
    )


def kernel(tok_seq, tok_type_ids, W_word, W_type, W_rel):
    types_col = tok_type_ids[:, :, None]
    wrel_pad = jnp.zeros((_KPAD, _D), jnp.float32).at[: 2 * _MAX_OFF + 2].set(W_rel)

    word_rows = _make_sc_word_gather()(W_word, tok_seq.reshape(-1))

    embd2 = pl.pallas_call(
        _embd2_body,
        grid=(_B, _S // _BI),
        in_specs=[
            pl.BlockSpec((_KPAD, _D), lambda b, i: (0, 0)),
            pl.BlockSpec(memory_space=pltpu.SMEM),
            pl.BlockSpec((1, _S, 1), lambda b, i: (b, 0, 0)),
        ],
        out_specs=pl.BlockSpec((1, _BI, _S, _D), lambda b, i: (b, i, 0, 0)),
        out_shape=jax.ShapeDtypeStruct((_B, _S, _S, _D), jnp.float32),
        scratch_shapes=[pltpu.VMEM((_BANKS * _TAB_ROWS, _D), jnp.float32)],
        compiler_params=pltpu.CompilerParams(
            dimension_semantics=("arbitrary", "arbitrary")),
    )(wrel_pad, tok_type_ids, types_col)

    embd1 = pl.pallas_call(
        _embd1_body,
        grid=(_B,),
        in_specs=[
            pl.BlockSpec((1, _S, _D), lambda b: (b, 0, 0)),
            pl.BlockSpec((1, _S, 1), lambda b: (b, 0, 0)),
            pl.BlockSpec((2, _D), lambda b: (0, 0)),
        ],
        out_specs=pl.BlockSpec((1, _S, _D), lambda b: (b, 0, 0)),
        out_shape=jax.ShapeDtypeStruct((_B, _S, _D), jnp.float32),
    )(word_rows.reshape(_B, _S, _D), types_col, W_type)

    return (embd1, embd2)


# speedup vs baseline: 6.9780x; 1.2067x over previous
import functools

import jax
import jax.numpy as jnp
from jax import lax
from jax.experimental import pallas as pl
from jax.experimental.pallas import tpu as pltpu
from jax.experimental.pallas import tpu_sc as plsc

_VOCAB = 100000
_D = 64
_MAX_OFF = 128
_EPS = 1e-12
_B, _S = 4, 512

_TAB_ROWS = 1040
_KPAD = 264
_BANKS = 8
_BI = 32


def _layernorm_rows(x):
    u = jnp.mean(x, axis=-1, keepdims=True)
    s = jnp.mean((x - u) ** 2, axis=-1, keepdims=True)
    return (x - u) / jnp.sqrt(s + _EPS)


def _embd2_body(wrel_ref, types_smem, tcol_ref, out_ref, tab_s):
    b = pl.program_id(0)
    ib = pl.program_id(1)

    @pl.when((b == 0) & (ib == 0))
    def _build_table():
        rr = lax.broadcasted_iota(jnp.int32, (_BANKS, _TAB_ROWS, _KPAD), 0)
        kk = lax.broadcasted_iota(jnp.int32, (_BANKS, _TAB_ROWS, _KPAD), 1)
        vv = lax.broadcasted_iota(jnp.int32, (_BANKS, _TAB_ROWS, _KPAD), 2)
        karg = kk + rr
        d = karg - (_S - 1)
        toepval = jnp.where(d >= 0, jnp.minimum(d, _MAX_OFF - 1),
                            jnp.maximum(d, -_MAX_OFF + 1) + 2 * _MAX_OFF)
        g = jnp.where(karg < 2 * _S - 1, toepval, 0)
        g = jnp.where(karg == 1024, _MAX_OFF, g)
        g = jnp.where(karg == 1025, 2 * _MAX_OFF, g)
        g = jnp.where(karg == 1026, 2 * _MAX_OFF + 1, g)
        onehot = (g == vv).astype(jnp.float32)
        onehot = onehot.reshape(_BANKS * _TAB_ROWS, _KPAD)
        ln_rel = _layernorm_rows(wrel_ref[...])
        tab_s[...] = jnp.dot(onehot, ln_rel,
                             precision=lax.Precision.HIGHEST,
                             preferred_element_type=jnp.float32)

    r128 = tab_s[1024:1025, :]
    for c in range(_BI):
        i = ib * _BI + c
        ti = types_smem[b, i]
        mask = tcol_ref[0] == ti
        s = _S - 1 - i
        r = lax.rem(s, 8)
        start = pl.multiple_of(r * _TAB_ROWS + (s - r), 8)
        win = tab_s[pl.ds(start, _S), :]
        base = jnp.where(mask, win, r128)

        if c == 0:
            @pl.when(ib == 0)
            def _first_row():
                jcol = lax.broadcasted_iota(jnp.int32, (_S, 1), 0)
                out_ref[0, 0] = jnp.where(jcol >= 1, tab_s[1025:1026, :], base)

            @pl.when(ib > 0)
            def _other_first():
                out_ref[0, 0] = base
                out_ref[0, 0, 0:1, :] = tab_s[1026:1027, :]
        else:
            out_ref[0, c] = base
            out_ref[0, c, 0:1, :] = tab_s[1026:1027, :]


def _embd1_body(rows_ref, tcol_ref, wtype_ref, out_ref):
    mask = tcol_ref[0] == 0
    tw = jnp.where(mask, wtype_ref[0:1, :], wtype_ref[1:2, :])
    out_ref[0] = _layernorm_rows(rows_ref[0] + tw)


_SC_CORES = 2
_SC_SUBCORES = 16
_NW = _SC_CORES * _SC_SUBCORES
_ROWS_PER_W = (_B * _S) // _NW


def _sc_word_gather_body(table_hbm, idx_hbm, out_hbm, idx_v, rows_v, sem):
    wid = lax.axis_index("s") * _SC_CORES + lax.axis_index("c")
    base = wid * _ROWS_PER_W
    pltpu.sync_copy(idx_hbm.at[pl.ds(base, _ROWS_PER_W)], idx_v)
    pltpu.async_copy(table_hbm.at[idx_v], rows_v, sem).wait()
    pltpu.sync_copy(rows_v, out_hbm.at[pl.ds(base, _ROWS_PER_W)])


def _make_sc_word_gather():
    return pl.kernel(
        _sc_word_gather_body,
        mesh=plsc.VectorSubcoreMesh(core_axis_name="c", subcore_axis_name="s"),
        out_type=jax.ShapeDtypeStruct((_B * _S, _D), jnp.float32),
        scratch_types=[
            pltpu.VMEM((_ROWS_PER_W,), jnp.int32),
            pltpu.VMEM((_ROWS_PER_W, _D), jnp.float32),
            pltpu.SemaphoreType.DMA,
        ],
        compiler_params=pltpu.CompilerParams(use_tc_tiling_on_sc=False),
    )


def kernel(tok_seq, tok_type_ids, W_word, W_type, W_rel):
    types_col = tok_type_ids[:, :, None]
    wrel_pad = jnp.zeros((_KPAD, _D), jnp.float32).at[: 2 * _MAX_OFF + 2].set(W_rel)

    word_rows = _make_sc_word_gather()(W_word, tok_seq.reshape(-1))

    embd2 = pl.pallas_call(
        _embd2_body,
        grid=(_B, _S // _BI),
        in_specs=[
            pl.BlockSpec((_KPAD, _D), lambda b, i: (0, 0)),
            pl.BlockSpec(memory_space=pltpu.SMEM),
            pl.BlockSpec((1, _S, 1), lambda b, i: (b, 0, 0)),
        ],
        out_specs=pl.BlockSpec((1, _BI, _S, _D), lambda b, i: (b, i, 0, 0)),
        out_shape=jax.ShapeDtypeStruct((_B, _S, _S, _D), jnp.float32),
        scratch_shapes=[pltpu.VMEM((_BANKS * _TAB_ROWS, _D), jnp.float32)],
        compiler_params=pltpu.CompilerParams(
            dimension_semantics=("arbitrary", "arbitrary")),
    )(wrel_pad, tok_type_ids, types_col)

    embd1 = pl.pallas_call(
        _embd1_body,
        grid=(_B,),
        in_specs=[
            pl.BlockSpec((1, _S, _D), lambda b: (b, 0, 0)),
            pl.BlockSpec((1, _S, 1), lambda b: (b, 0, 0)),
            pl.BlockSpec((2, _D), lambda b: (0, 0)),
        ],
        out_specs=pl.BlockSpec((1, _S, _D), lambda b: (b, 0, 0)),
        out_shape=jax.ShapeDtypeStruct((_B, _S, _D), jnp.float32),
    )(word_rows.reshape(_B, _S, _D), types_col, W_type)

    return (embd1, embd2)


# speedup vs baseline: 7.0736x; 1.0137x over previous
import functools

import jax
import jax.numpy as jnp
from jax import lax
from jax.experimental import pallas as pl
from jax.experimental.pallas import tpu as pltpu
from jax.experimental.pallas import tpu_sc as plsc

_VOCAB = 100000
_D = 64
_MAX_OFF = 128
_EPS = 1e-12
_B, _S = 4, 512

_TAB_ROWS = 1040
_KPAD = 264
_BANKS = 8
_BI = 64


def _layernorm_rows(x):
    u = jnp.mean(x, axis=-1, keepdims=True)
    s = jnp.mean((x - u) ** 2, axis=-1, keepdims=True)
    return (x - u) / jnp.sqrt(s + _EPS)


def _embd2_body(wrel_ref, types_smem, tcol_ref, out_ref, tab_s):
    b = pl.program_id(0)
    ib = pl.program_id(1)

    @pl.when((b == 0) & (ib == 0))
    def _build_table():
        rr = lax.broadcasted_iota(jnp.int32, (_BANKS, _TAB_ROWS, _KPAD), 0)
        kk = lax.broadcasted_iota(jnp.int32, (_BANKS, _TAB_ROWS, _KPAD), 1)
        vv = lax.broadcasted_iota(jnp.int32, (_BANKS, _TAB_ROWS, _KPAD), 2)
        karg = kk + rr
        d = karg - (_S - 1)
        toepval = jnp.where(d >= 0, jnp.minimum(d, _MAX_OFF - 1),
                            jnp.maximum(d, -_MAX_OFF + 1) + 2 * _MAX_OFF)
        g = jnp.where(karg < 2 * _S - 1, toepval, 0)
        g = jnp.where(karg == 1024, _MAX_OFF, g)
        g = jnp.where(karg == 1025, 2 * _MAX_OFF, g)
        g = jnp.where(karg == 1026, 2 * _MAX_OFF + 1, g)
        onehot = (g == vv).astype(jnp.float32)
        onehot = onehot.reshape(_BANKS * _TAB_ROWS, _KPAD)
        ln_rel = _layernorm_rows(wrel_ref[...])
        tab_s[...] = jnp.dot(onehot, ln_rel,
                             precision=lax.Precision.HIGHEST,
                             preferred_element_type=jnp.float32)

    r128 = tab_s[1024:1025, :]
    for c in range(_BI):
        i = ib * _BI + c
        ti = types_smem[b, i]
        mask = tcol_ref[0] == ti
        s = _S - 1 - i
        r = lax.rem(s, 8)
        start = pl.multiple_of(r * _TAB_ROWS + (s - r), 8)
        win = tab_s[pl.ds(start, _S), :]
        base = jnp.where(mask, win, r128)

        if c == 0:
            @pl.when(ib == 0)
            def _first_row():
                jcol = lax.broadcasted_iota(jnp.int32, (_S, 1), 0)
                out_ref[0, 0] = jnp.where(jcol >= 1, tab_s[1025:1026, :], base)

            @pl.when(ib > 0)
            def _other_first():
                out_ref[0, 0] = base
                out_ref[0, 0, 0:1, :] = tab_s[1026:1027, :]
        else:
            out_ref[0, c] = base
            out_ref[0, c, 0:1, :] = tab_s[1026:1027, :]


def _embd1_body(rows_ref, tcol_ref, wtype_ref, out_ref):
    mask = tcol_ref[0] == 0
    tw = jnp.where(mask, wtype_ref[0:1, :], wtype_ref[1:2, :])
    out_ref[0] = _layernorm_rows(rows_ref[0] + tw)


_SC_CORES = 2
_SC_SUBCORES = 16
_NW = _SC_CORES * _SC_SUBCORES
_ROWS_PER_W = (_B * _S) // _NW


def _sc_word_gather_body(table_hbm, idx_hbm, out_hbm, idx_v, rows_v, sem):
    wid = lax.axis_index("s") * _SC_CORES + lax.axis_index("c")
    base = wid * _ROWS_PER_W
    pltpu.sync_copy(idx_hbm.at[pl.ds(base, _ROWS_PER_W)], idx_v)
    pltpu.async_copy(table_hbm.at[idx_v], rows_v, sem).wait()
    pltpu.sync_copy(rows_v, out_hbm.at[pl.ds(base, _ROWS_PER_W)])


def _make_sc_word_gather():
    return pl.kernel(
        _sc_word_gather_body,
        mesh=plsc.VectorSubcoreMesh(core_axis_name="c", subcore_axis_name="s"),
        out_type=jax.ShapeDtypeStruct((_B * _S, _D), jnp.float32),
        scratch_types=[
            pltpu.VMEM((_ROWS_PER_W,), jnp.int32),
            pltpu.VMEM((_ROWS_PER_W, _D), jnp.float32),
            pltpu.SemaphoreType.DMA,
        ],
        compiler_params=pltpu.CompilerParams(use_tc_tiling_on_sc=False),
    )


def kernel(tok_seq, tok_type_ids, W_word, W_type, W_rel):
    types_col = tok_type_ids[:, :, None]
    wrel_pad = jnp.zeros((_KPAD, _D), jnp.float32).at[: 2 * _MAX_OFF + 2].set(W_rel)

    word_rows = _make_sc_word_gather()(W_word, tok_seq.reshape(-1))

    embd2 = pl.pallas_call(
        _embd2_body,
        grid=(_B, _S // _BI),
        in_specs=[
            pl.BlockSpec((_KPAD, _D), lambda b, i: (0, 0)),
            pl.BlockSpec(memory_space=pltpu.SMEM),
            pl.BlockSpec((1, _S, 1), lambda b, i: (b, 0, 0)),
        ],
        out_specs=pl.BlockSpec((1, _BI, _S, _D), lambda b, i: (b, i, 0, 0)),
        out_shape=jax.ShapeDtypeStruct((_B, _S, _S, _D), jnp.float32),
        scratch_shapes=[pltpu.VMEM((_BANKS * _TAB_ROWS, _D), jnp.float32)],
        compiler_params=pltpu.CompilerParams(
            dimension_semantics=("arbitrary", "arbitrary"),
            vmem_limit_bytes=60000 * 1024),
    )(wrel_pad, tok_type_ids, types_col)

    embd1 = pl.pallas_call(
        _embd1_body,
        grid=(_B,),
        in_specs=[
            pl.BlockSpec((1, _S, _D), lambda b: (b, 0, 0)),
            pl.BlockSpec((1, _S, 1), lambda b: (b, 0, 0)),
            pl.BlockSpec((2, _D), lambda b: (0, 0)),
        ],
        out_specs=pl.BlockSpec((1, _S, _D), lambda b: (b, 0, 0)),
        out_shape=jax.ShapeDtypeStruct((_B, _S, _D), jnp.float32),
    )(word_rows.reshape(_B, _S, _D), types_col, W_type)

    return (embd1, embd2)
